# baseline (speedup 1.0000x reference)
import math

import jax
import jax.numpy as jnp
import numpy as np
from jax import lax
from jax.experimental import pallas as pl
from jax.experimental.pallas import tpu as pltpu
from jax.experimental.pallas import tpu_sc as plsc

N_LEVELS = 16
F = 2
LOG2_T = 19
T = 1 << LOG2_T
MASK = T - 1
BASE_RES = 16
PER_LEVEL_SCALE = 1.5
P1 = int(np.uint32(2654435761).view(np.int32))
P2 = int(np.uint32(805459861).view(np.int32))
STEP_LENGTH = math.sqrt(3) / 1024

NC, NS = 2, 16
NW = NC * NS


def _res(l):
    return int(math.floor(BASE_RES * (PER_LEVEL_SCALE ** l)))


def _is_dense(l):
    return (_res(l) + 1) ** 3 <= T


def _sc_hash_encode(px, py, pz, tabf):
    N = px.shape[0]
    npw = N // NW
    B = 1024 if npw % 1024 == 0 else npw
    G = B // 16
    ND = (8 * B) // 128
    n_chunks = npw // B

    mesh = plsc.VectorSubcoreMesh(core_axis_name="c", subcore_axis_name="s",
                                  num_cores=NC, num_subcores=NS)

    def body(px_h, py_h, pz_h, tab_h, out_h, xb, yb, zb, idxb, wb, rowsb,
             featb, sem):
        wid = lax.axis_index("s") * NC + lax.axis_index("c")

        def chunk_body(ci, _):
            base = wid * npw + ci * B
            pltpu.sync_copy(px_h.at[pl.ds(base, B)], xb)
            pltpu.sync_copy(py_h.at[pl.ds(base, B)], yb)
            pltpu.sync_copy(pz_h.at[pl.ds(base, B)], zb)

            for l in range(N_LEVELS):
                res = _res(l)
                dense = _is_dense(l)
                s = res + 1

                def gen(j, _, l=l, res=res, dense=dense, s=s):
                    off = j * 16
                    x = xb[pl.ds(off, 16)]
                    y = yb[pl.ds(off, 16)]
                    z = zb[pl.ds(off, 16)]
                    posx = x * float(res)
                    posy = y * float(res)
                    posz = z * float(res)
                    pix = posx.astype(jnp.int32)
                    piy = posy.astype(jnp.int32)
                    piz = posz.astype(jnp.int32)
                    fx = posx - pix.astype(jnp.float32)
                    fy = posy - piy.astype(jnp.float32)
                    fz = posz - piz.astype(jnp.float32)
                    if dense:
                        tx = (pix, pix + 1)
                        ty = (piy * s, piy * s + s)
                        tz = (piz * (s * s), piz * (s * s) + s * s)
                    else:
                        tx = (pix, pix + 1)
                        ty = (piy * P1, piy * P1 + P1)
                        tz = (piz * P2, piz * P2 + P2)
                    wx = (1.0 - fx, fx)
                    wy = (1.0 - fy, fy)
                    wz = (1.0 - fz, fz)
                    r8 = j // 8
                    c8 = (j % 8) * 16
                    wxy = [wx[0] * wy[0], wx[1] * wy[0], wx[0] * wy[1],
                           wx[1] * wy[1]]
                    for c in range(8):
                        bx, by, bz = c & 1, (c >> 1) & 1, (c >> 2) & 1
                        if dense:
                            idx = tx[bx] + ty[by] + tz[bz]
                        else:
                            idx = (tx[bx] ^ ty[by] ^ tz[bz]) & MASK
                        e0 = idx * 2 + (2 * l * T)
                        w = wxy[c & 3] * wz[bz]
                        idxb[c * (B // 128) + r8, pl.ds(c8, 16)] = e0
                        idxb[ND + c * (B // 128) + r8, pl.ds(c8, 16)] = e0 + 1
                        wb[c, pl.ds(off, 16)] = w
                    return 0

                lax.fori_loop(0, G, gen, 0)

                def fire(j, _):
                    pltpu.make_async_copy(
                        tab_h.at[idxb.at[j]],
                        rowsb.at[pl.ds(j * 128, 128)], sem).start()
                    return 0

                lax.fori_loop(0, 2 * ND, fire, 0)

                def drain(j, _):
                    pltpu.make_async_copy(
                        tab_h.at[idxb.at[j]],
                        rowsb.at[pl.ds(j * 128, 128)], sem).wait()
                    return 0

                lax.fori_loop(0, 2 * ND, drain, 0)

                def acc(j, _, l=l):
                    off = j * 16
                    f0 = jnp.zeros((16,), jnp.float32)
                    f1 = jnp.zeros((16,), jnp.float32)
                    for c in range(8):
                        g0 = rowsb[pl.ds(c * B + off, 16)]
                        g1 = rowsb[pl.ds(8 * B + c * B + off, 16)]
                        w = wb[c, pl.ds(off, 16)]
                        f0 = f0 + w * g0
                        f1 = f1 + w * g1
                    featb[2 * l, pl.ds(off, 16)] = f0
                    featb[2 * l + 1, pl.ds(off, 16)] = f1
                    return 0

                lax.fori_loop(0, G, acc, 0)

            pltpu.sync_copy(featb, out_h.at[:, pl.ds(base, B)])
            return 0

        lax.fori_loop(0, n_chunks, chunk_body, 0)

    run = pl.kernel(
        body,
        out_type=jax.ShapeDtypeStruct((2 * N_LEVELS, N), jnp.float32),
        mesh=mesh,
        scratch_types=[
            pltpu.VMEM((B,), jnp.float32),
            pltpu.VMEM((B,), jnp.float32),
            pltpu.VMEM((B,), jnp.float32),
            pltpu.VMEM((2 * ND, 128), jnp.int32),
            pltpu.VMEM((8, B), jnp.float32),
            pltpu.VMEM((16 * B,), jnp.float32),
            pltpu.VMEM((2 * N_LEVELS, B), jnp.float32),
            pltpu.SemaphoreType.DMA,
        ],
    )
    return run(px, py, pz, tabf)


def _tc_mlp(feats, dirT, w1s, w2s, w1r, w2r, w3r):
    N = feats.shape[1]
    NB = 2048 if N % 2048 == 0 else N

    def body(f_ref, d_ref, w1s_ref, w2s_ref, w1r_ref, w2r_ref, w3r_ref,
             o_ref):
        f = f_ref[...]
        hp = jax.lax.dot_general(
            w1s_ref[...], f, (((1,), (0,)), ((), ())),
            precision=lax.Precision.HIGHEST,
            preferred_element_type=jnp.float32)
        h = jnp.maximum(hp, 0.0)
        hf = jax.lax.dot_general(
            w2s_ref[...], h, (((1,), (0,)), ((), ())),
            precision=lax.Precision.HIGHEST,
            preferred_element_type=jnp.float32)
        alpha = 1.0 - jnp.exp(-jnp.exp(hf[0:1, :]) * STEP_LENGTH)

        dd = (d_ref[...] + 1.0) * 0.5 * 2.0 - 1.0
        x, y, z = dd[0:1, :], dd[1:2, :], dd[2:3, :]
        xy, xz, yz = x * y, x * z, y * z
        x2, y2, z2 = x * x, y * y, z * z
        sh = jnp.concatenate([
            jnp.full_like(x, 0.28209479177387814),
            -0.48860251190291987 * y,
            0.48860251190291987 * z,
            -0.48860251190291987 * x,
            1.0925484305920792 * xy,
            -1.0925484305920792 * yz,
            0.94617469575755997 * z2 - 0.31539156525251999,
            -1.0925484305920792 * xz,
            0.54627421529603959 * x2 - 0.54627421529603959 * y2,
            0.59004358992664352 * y * (-3.0 * x2 + y2),
            2.8906114426405538 * xy * z,
            0.45704579946446572 * y * (1.0 - 5.0 * z2),
            0.3731763325901154 * z * (5.0 * z2 - 3.0),
            0.45704579946446572 * x * (1.0 - 5.0 * z2),
            1.4453057213202769 * z * (x2 - y2),
            0.59004358992664352 * x * (x2 - 3.0 * y2),
        ], axis=0)

        feats2 = jnp.concatenate([hf, sh], axis=0)
        r = jnp.maximum(jax.lax.dot_general(
            w1r_ref[...], feats2, (((1,), (0,)), ((), ())),
            precision=lax.Precision.HIGHEST,
            preferred_element_type=jnp.float32), 0.0)
        r = jnp.maximum(jax.lax.dot_general(
            w2r_ref[...], r, (((1,), (0,)), ((), ())),
            precision=lax.Precision.HIGHEST,
            preferred_element_type=jnp.float32), 0.0)
        rgb = jax.nn.sigmoid(jax.lax.dot_general(
            w3r_ref[...], r, (((1,), (0,)), ((), ())),
            precision=lax.Precision.HIGHEST,
            preferred_element_type=jnp.float32))
        o_ref[...] = jnp.concatenate([rgb, alpha], axis=0)

    return pl.pallas_call(
        body,
        grid=(N // NB,),
        in_specs=[
            pl.BlockSpec((2 * N_LEVELS, NB), lambda i: (0, i)),
            pl.BlockSpec((3, NB), lambda i: (0, i)),
            pl.BlockSpec((64, 32), lambda i: (0, 0)),
            pl.BlockSpec((16, 64), lambda i: (0, 0)),
            pl.BlockSpec((64, 32), lambda i: (0, 0)),
            pl.BlockSpec((64, 64), lambda i: (0, 0)),
            pl.BlockSpec((3, 64), lambda i: (0, 0)),
        ],
        out_specs=pl.BlockSpec((4, NB), lambda i: (0, i)),
        out_shape=jax.ShapeDtypeStruct((4, N), jnp.float32),
    )(feats, dirT, w1s, w2s, w1r, w2r, w3r)


def kernel(position, direction, table, w_sig1, w_sig2, w_rgb1, w_rgb2,
           w_rgb3):
    px = position[:, 0]
    py = position[:, 1]
    pz = position[:, 2]
    tabf = table.reshape(N_LEVELS * T * F)
    feats = _sc_hash_encode(px, py, pz, tabf)
    out4 = _tc_mlp(feats, direction.T, w_sig1.T, w_sig2.T, w_rgb1.T,
                   w_rgb2.T, w_rgb3.T)
    rgbs = out4[:3].T
    alphas = out4[3]
    return (rgbs, alphas)


# speedup vs baseline: 2.6285x; 2.6285x over previous
import math

import jax
import jax.numpy as jnp
import numpy as np
from jax import lax
from jax.experimental import pallas as pl
from jax.experimental.pallas import tpu as pltpu
from jax.experimental.pallas import tpu_sc as plsc

N_LEVELS = 16
F = 2
LOG2_T = 19
T = 1 << LOG2_T
MASK = T - 1
BASE_RES = 16
PER_LEVEL_SCALE = 1.5
P1 = int(np.uint32(2654435761).view(np.int32))
P2 = int(np.uint32(805459861).view(np.int32))
STEP_LENGTH = math.sqrt(3) / 1024

NC, NS = 2, 16
NW = NC * NS


def _res(l):
    return int(math.floor(BASE_RES * (PER_LEVEL_SCALE ** l)))


def _is_dense(l):
    return (_res(l) + 1) ** 3 <= T


def _sc_hash_encode(px, py, pz, tabf):
    N = px.shape[0]
    npw = N // NW
    B = 1024 if npw % 1024 == 0 else npw
    G = B // 16
    ND = (8 * B) // 128
    n_chunks = npw // B

    mesh = plsc.VectorSubcoreMesh(core_axis_name="c", subcore_axis_name="s",
                                  num_cores=NC, num_subcores=NS)

    def body(px_h, py_h, pz_h, tab_h, out_h, xb, yb, zb, idxb, wb, rowsb,
             featb, sem):
        wid = lax.axis_index("s") * NC + lax.axis_index("c")

        def chunk_body(ci, _):
            base = wid * npw + ci * B
            pltpu.sync_copy(px_h.at[pl.ds(base, B)], xb)
            pltpu.sync_copy(py_h.at[pl.ds(base, B)], yb)
            pltpu.sync_copy(pz_h.at[pl.ds(base, B)], zb)

            for l in range(N_LEVELS):
                res = _res(l)
                dense = _is_dense(l)
                s = res + 1

                def gen(j, _, l=l, res=res, dense=dense, s=s):
                    off = j * 16
                    x = xb[pl.ds(off, 16)]
                    y = yb[pl.ds(off, 16)]
                    z = zb[pl.ds(off, 16)]
                    posx = x * float(res)
                    posy = y * float(res)
                    posz = z * float(res)
                    pix = posx.astype(jnp.int32)
                    piy = posy.astype(jnp.int32)
                    piz = posz.astype(jnp.int32)
                    fx = posx - pix.astype(jnp.float32)
                    fy = posy - piy.astype(jnp.float32)
                    fz = posz - piz.astype(jnp.float32)
                    if dense:
                        tx = (pix, pix + 1)
                        ty = (piy * s, piy * s + s)
                        tz = (piz * (s * s), piz * (s * s) + s * s)
                    else:
                        tx = (pix, pix + 1)
                        ty = (piy * P1, piy * P1 + P1)
                        tz = (piz * P2, piz * P2 + P2)
                    wx = (1.0 - fx, fx)
                    wy = (1.0 - fy, fy)
                    wz = (1.0 - fz, fz)
                    r8 = j // 8
                    c8 = (j % 8) * 16
                    wxy = [wx[0] * wy[0], wx[1] * wy[0], wx[0] * wy[1],
                           wx[1] * wy[1]]
                    for c in range(8):
                        bx, by, bz = c & 1, (c >> 1) & 1, (c >> 2) & 1
                        if dense:
                            idx = tx[bx] + ty[by] + tz[bz]
                        else:
                            idx = (tx[bx] ^ ty[by] ^ tz[bz]) & MASK
                        e0 = (((idx >> 7) << 8) | (idx & 127)) + (2 * l * T)
                        w = wxy[c & 3] * wz[bz]
                        idxb[c * (B // 128) + r8, pl.ds(c8, 16)] = e0
                        idxb[ND + c * (B // 128) + r8, pl.ds(c8, 16)] = e0 + 128
                        wb[c, pl.ds(off, 16)] = w
                    return 0

                lax.fori_loop(0, G, gen, 0)

                def fire(j, _):
                    pltpu.make_async_copy(
                        tab_h.at[idxb.at[j]],
                        rowsb.at[pl.ds(j * 128, 128)], sem).start()
                    return 0

                lax.fori_loop(0, 2 * ND, fire, 0)

                def drain(j, _):
                    pltpu.make_async_copy(
                        tab_h.at[idxb.at[j]],
                        rowsb.at[pl.ds(j * 128, 128)], sem).wait()
                    return 0

                lax.fori_loop(0, 2 * ND, drain, 0)

                def acc(j, _, l=l):
                    off = j * 16
                    f0 = jnp.zeros((16,), jnp.float32)
                    f1 = jnp.zeros((16,), jnp.float32)
                    for c in range(8):
                        g0 = rowsb[pl.ds(c * B + off, 16)]
                        g1 = rowsb[pl.ds(8 * B + c * B + off, 16)]
                        w = wb[c, pl.ds(off, 16)]
                        f0 = f0 + w * g0
                        f1 = f1 + w * g1
                    featb[2 * l, pl.ds(off, 16)] = f0
                    featb[2 * l + 1, pl.ds(off, 16)] = f1
                    return 0

                lax.fori_loop(0, G, acc, 0)

            pltpu.sync_copy(featb, out_h.at[:, pl.ds(base, B)])
            return 0

        lax.fori_loop(0, n_chunks, chunk_body, 0)

    run = pl.kernel(
        body,
        out_type=jax.ShapeDtypeStruct((2 * N_LEVELS, N), jnp.float32),
        mesh=mesh,
        scratch_types=[
            pltpu.VMEM((B,), jnp.float32),
            pltpu.VMEM((B,), jnp.float32),
            pltpu.VMEM((B,), jnp.float32),
            pltpu.VMEM((2 * ND, 128), jnp.int32),
            pltpu.VMEM((8, B), jnp.float32),
            pltpu.VMEM((16 * B,), jnp.float32),
            pltpu.VMEM((2 * N_LEVELS, B), jnp.float32),
            pltpu.SemaphoreType.DMA,
        ],
    )
    return run(px, py, pz, tabf)


def _tc_mlp(feats, dirT, w1s, w2s, w1r, w2r, w3r):
    N = feats.shape[1]
    NB = 2048 if N % 2048 == 0 else N

    def body(f_ref, d_ref, w1s_ref, w2s_ref, w1r_ref, w2r_ref, w3r_ref,
             o_ref):
        f = f_ref[...]
        hp = jax.lax.dot_general(
            w1s_ref[...], f, (((1,), (0,)), ((), ())),
            precision=lax.Precision.HIGHEST,
            preferred_element_type=jnp.float32)
        h = jnp.maximum(hp, 0.0)
        hf = jax.lax.dot_general(
            w2s_ref[...], h, (((1,), (0,)), ((), ())),
            precision=lax.Precision.HIGHEST,
            preferred_element_type=jnp.float32)
        alpha = 1.0 - jnp.exp(-jnp.exp(hf[0:1, :]) * STEP_LENGTH)

        dd = (d_ref[...] + 1.0) * 0.5 * 2.0 - 1.0
        x, y, z = dd[0:1, :], dd[1:2, :], dd[2:3, :]
        xy, xz, yz = x * y, x * z, y * z
        x2, y2, z2 = x * x, y * y, z * z
        sh = jnp.concatenate([
            jnp.full_like(x, 0.28209479177387814),
            -0.48860251190291987 * y,
            0.48860251190291987 * z,
            -0.48860251190291987 * x,
            1.0925484305920792 * xy,
            -1.0925484305920792 * yz,
            0.94617469575755997 * z2 - 0.31539156525251999,
            -1.0925484305920792 * xz,
            0.54627421529603959 * x2 - 0.54627421529603959 * y2,
            0.59004358992664352 * y * (-3.0 * x2 + y2),
            2.8906114426405538 * xy * z,
            0.45704579946446572 * y * (1.0 - 5.0 * z2),
            0.3731763325901154 * z * (5.0 * z2 - 3.0),
            0.45704579946446572 * x * (1.0 - 5.0 * z2),
            1.4453057213202769 * z * (x2 - y2),
            0.59004358992664352 * x * (x2 - 3.0 * y2),
        ], axis=0)

        feats2 = jnp.concatenate([hf, sh], axis=0)
        r = jnp.maximum(jax.lax.dot_general(
            w1r_ref[...], feats2, (((1,), (0,)), ((), ())),
            precision=lax.Precision.HIGHEST,
            preferred_element_type=jnp.float32), 0.0)
        r = jnp.maximum(jax.lax.dot_general(
            w2r_ref[...], r, (((1,), (0,)), ((), ())),
            precision=lax.Precision.HIGHEST,
            preferred_element_type=jnp.float32), 0.0)
        rgb = jax.nn.sigmoid(jax.lax.dot_general(
            w3r_ref[...], r, (((1,), (0,)), ((), ())),
            precision=lax.Precision.HIGHEST,
            preferred_element_type=jnp.float32))
        o_ref[...] = jnp.concatenate([rgb, alpha], axis=0)

    return pl.pallas_call(
        body,
        grid=(N // NB,),
        in_specs=[
            pl.BlockSpec((2 * N_LEVELS, NB), lambda i: (0, i)),
            pl.BlockSpec((3, NB), lambda i: (0, i)),
            pl.BlockSpec((64, 32), lambda i: (0, 0)),
            pl.BlockSpec((16, 64), lambda i: (0, 0)),
            pl.BlockSpec((64, 32), lambda i: (0, 0)),
            pl.BlockSpec((64, 64), lambda i: (0, 0)),
            pl.BlockSpec((3, 64), lambda i: (0, 0)),
        ],
        out_specs=pl.BlockSpec((4, NB), lambda i: (0, i)),
        out_shape=jax.ShapeDtypeStruct((4, N), jnp.float32),
    )(feats, dirT, w1s, w2s, w1r, w2r, w3r)


def kernel(position, direction, table, w_sig1, w_sig2, w_rgb1, w_rgb2,
           w_rgb3):
    px = position[:, 0]
    py = position[:, 1]
    pz = position[:, 2]
    tabf = table.reshape(N_LEVELS, T // 128, 128, F)
    tabf = tabf.transpose(0, 1, 3, 2).reshape(N_LEVELS * T * F)
    feats = _sc_hash_encode(px, py, pz, tabf)
    out4 = _tc_mlp(feats, direction.T, w_sig1.T, w_sig2.T, w_rgb1.T,
                   w_rgb2.T, w_rgb3.T)
    rgbs = out4[:3].T
    alphas = out4[3]
    return (rgbs, alphas)


# speedup vs baseline: 3.3261x; 1.2654x over previous
import math

import jax
import jax.numpy as jnp
import numpy as np
from jax import lax
from jax.experimental import pallas as pl
from jax.experimental.pallas import tpu as pltpu
from jax.experimental.pallas import tpu_sc as plsc

N_LEVELS = 16
F = 2
LOG2_T = 19
T = 1 << LOG2_T
MASK = T - 1
BASE_RES = 16
PER_LEVEL_SCALE = 1.5
P1 = int(np.uint32(2654435761).view(np.int32))
P2 = int(np.uint32(805459861).view(np.int32))
STEP_LENGTH = math.sqrt(3) / 1024

NC, NS = 2, 16
NW = NC * NS


def _res(l):
    return int(math.floor(BASE_RES * (PER_LEVEL_SCALE ** l)))


def _is_dense(l):
    return (_res(l) + 1) ** 3 <= T


_VLEVELS = (0, 1)
_VCHUNKS = tuple((_res(l) + 1) ** 3 // 128 + 1 for l in _VLEVELS)
_VSIZE = tuple(c * 256 for c in _VCHUNKS)


def _sc_hash_encode(px, py, pz, tabf):
    N = px.shape[0]
    npw = N // NW
    B = 512 if npw % 512 == 0 else npw
    G = B // 16
    NB128 = B // 128
    NF = 8 * NB128
    n_chunks = npw // B

    mesh = plsc.VectorSubcoreMesh(core_axis_name="c", subcore_axis_name="s",
                                  num_cores=NC, num_subcores=NS)

    def _gen(l, j, xb, yb, zb, idxb, wb):
        res = _res(l)
        dense = _is_dense(l)
        s = res + 1
        off = j * 16
        x = xb[pl.ds(off, 16)]
        y = yb[pl.ds(off, 16)]
        z = zb[pl.ds(off, 16)]
        posx = x * float(res)
        posy = y * float(res)
        posz = z * float(res)
        pix = posx.astype(jnp.int32)
        piy = posy.astype(jnp.int32)
        piz = posz.astype(jnp.int32)
        fx = posx - pix.astype(jnp.float32)
        fy = posy - piy.astype(jnp.float32)
        fz = posz - piz.astype(jnp.float32)
        if dense:
            tx = (pix, pix + 1)
            ty = (piy * s, piy * s + s)
            tz = (piz * (s * s), piz * (s * s) + s * s)
        else:
            tx = (pix, pix + 1)
            ty = (piy * P1, piy * P1 + P1)
            tz = (piz * P2, piz * P2 + P2)
        wx = (1.0 - fx, fx)
        wy = (1.0 - fy, fy)
        wz = (1.0 - fz, fz)
        wxy = (wx[0] * wy[0], wx[1] * wy[0], wx[0] * wy[1], wx[1] * wy[1])
        base = 2 * l * T
        out = []
        for c in range(8):
            bx, bz = c & 1, (c >> 2) & 1
            if dense:
                idx = tx[bx] + ty[(c >> 1) & 1] + tz[bz]
            else:
                idx = (tx[bx] ^ ty[(c >> 1) & 1] ^ tz[bz]) & MASK
            e0 = (((idx >> 7) << 8) | (idx & 127)) + base
            w = wxy[c & 3] * wz[bz]
            out.append((e0, w))
        r8 = j // 8
        c8 = (j % 8) * 16
        for c, (e0, w) in enumerate(out):
            idxb[0, c * (B // 128) + r8, pl.ds(c8, 16)] = e0
            idxb[1, c * (B // 128) + r8, pl.ds(c8, 16)] = e0 + 128
            wb[c, pl.ds(off, 16)] = w

    def body(px_h, py_h, pz_h, tab_h, out_h, xb, yb, zb, t0v, t1v,
             idxb0, idxb1, wb0, wb1, rowsb0, rowsb1, featb, sem):
        wid = lax.axis_index("s") * NC + lax.axis_index("c")
        pltpu.sync_copy(tab_h.at[pl.ds(0, _VSIZE[0])], t0v)
        pltpu.sync_copy(tab_h.at[pl.ds(2 * T, _VSIZE[1])], t1v)
        bufs = ((idxb0, wb0, rowsb0), (idxb1, wb1, rowsb1))

        def fire(l, wait):
            idxb, _, rowsb = bufs[l & 1]

            def go(k, _):
                a = pltpu.make_async_copy(
                    tab_h.at[idxb.at[0, k]],
                    rowsb.at[0, k], sem)
                b = pltpu.make_async_copy(
                    tab_h.at[idxb.at[1, k]],
                    rowsb.at[1, k], sem)
                if wait:
                    a.wait()
                    b.wait()
                else:
                    a.start()
                    b.start()
                return 0

            lax.fori_loop(0, NF, go, 0)

        def gen_level(l):
            idxb, wb, _ = bufs[l & 1]

            def go(j, _):
                _gen(l, j, xb, yb, zb, idxb, wb)
                return 0

            lax.fori_loop(0, G, go, 0)

        def acc_level(l):
            _, wb, rowsb = bufs[l & 1]

            def go(j, _):
                off = j * 16
                f0 = jnp.zeros((16,), jnp.float32)
                f1 = jnp.zeros((16,), jnp.float32)
                r8 = j // 8
                c8 = (j % 8) * 16
                for c in range(8):
                    g0 = rowsb[0, c * (B // 128) + r8, pl.ds(c8, 16)]
                    g1 = rowsb[1, c * (B // 128) + r8, pl.ds(c8, 16)]
                    w = wb[c, pl.ds(off, 16)]
                    f0 = f0 + w * g0
                    f1 = f1 + w * g1
                featb[2 * l, pl.ds(off, 16)] = f0
                featb[2 * l + 1, pl.ds(off, 16)] = f1
                return 0

            lax.fori_loop(0, G, go, 0)

        def vmem_level(l, tv):
            res = _res(l)
            s = res + 1

            def go(j, _):
                off = j * 16
                x = xb[pl.ds(off, 16)]
                y = yb[pl.ds(off, 16)]
                z = zb[pl.ds(off, 16)]
                posx = x * float(res)
                posy = y * float(res)
                posz = z * float(res)
                pix = posx.astype(jnp.int32)
                piy = posy.astype(jnp.int32)
                piz = posz.astype(jnp.int32)
                fx = posx - pix.astype(jnp.float32)
                fy = posy - piy.astype(jnp.float32)
                fz = posz - piz.astype(jnp.float32)
                tx = (pix, pix + 1)
                ty = (piy * s, piy * s + s)
                tz = (piz * (s * s), piz * (s * s) + s * s)
                wx = (1.0 - fx, fx)
                wy = (1.0 - fy, fy)
                wz = (1.0 - fz, fz)
                wxy = (wx[0] * wy[0], wx[1] * wy[0], wx[0] * wy[1],
                       wx[1] * wy[1])
                f0 = jnp.zeros((16,), jnp.float32)
                f1 = jnp.zeros((16,), jnp.float32)
                for c in range(8):
                    bx, bz = c & 1, (c >> 2) & 1
                    idx = tx[bx] + ty[(c >> 1) & 1] + tz[bz]
                    e0 = ((idx >> 7) << 8) | (idx & 127)
                    g0 = plsc.load_gather(tv, [e0])
                    g1 = plsc.load_gather(tv, [e0 + 128])
                    w = wxy[c & 3] * wz[bz]
                    f0 = f0 + w * g0
                    f1 = f1 + w * g1
                featb[2 * l, pl.ds(off, 16)] = f0
                featb[2 * l + 1, pl.ds(off, 16)] = f1
                return 0

            lax.fori_loop(0, G, go, 0)

        def chunk_body(ci, _):
            base = wid * npw + ci * B
            pltpu.sync_copy(px_h.at[pl.ds(base, B)], xb)
            pltpu.sync_copy(py_h.at[pl.ds(base, B)], yb)
            pltpu.sync_copy(pz_h.at[pl.ds(base, B)], zb)

            gen_level(2)
            fire(2, False)
            vmem_level(0, t0v)
            vmem_level(1, t1v)
            for l in range(3, N_LEVELS):
                gen_level(l)
                fire(l - 1, True)
                fire(l, False)
                acc_level(l - 1)
            fire(N_LEVELS - 1, True)
            acc_level(N_LEVELS - 1)

            pltpu.sync_copy(featb, out_h.at[:, pl.ds(base, B)])
            return 0

        lax.fori_loop(0, n_chunks, chunk_body, 0)

    run = pl.kernel(
        body,
        out_type=jax.ShapeDtypeStruct((2 * N_LEVELS, N), jnp.float32),
        mesh=mesh,
        compiler_params=pltpu.CompilerParams(needs_layout_passes=False),
        scratch_types=[
            pltpu.VMEM((B,), jnp.float32),
            pltpu.VMEM((B,), jnp.float32),
            pltpu.VMEM((B,), jnp.float32),
            pltpu.VMEM((_VSIZE[0],), jnp.float32),
            pltpu.VMEM((_VSIZE[1],), jnp.float32),
            pltpu.VMEM((2, 8 * (B // 128), 128), jnp.int32),
            pltpu.VMEM((2, 8 * (B // 128), 128), jnp.int32),
            pltpu.VMEM((8, B), jnp.float32),
            pltpu.VMEM((8, B), jnp.float32),
            pltpu.VMEM((2, 8 * (B // 128), 128), jnp.float32),
            pltpu.VMEM((2, 8 * (B // 128), 128), jnp.float32),
            pltpu.VMEM((2 * N_LEVELS, B), jnp.float32),
            pltpu.SemaphoreType.DMA,
        ],
    )
    return run(px, py, pz, tabf)


def _tc_mlp(feats, dirT, w1s, w2s, w1r, w2r, w3r):
    N = feats.shape[1]
    NB = 2048 if N % 2048 == 0 else N

    def body(f_ref, d_ref, w1s_ref, w2s_ref, w1r_ref, w2r_ref, w3r_ref,
             o_ref):
        f = f_ref[...]
        hp = jax.lax.dot_general(
            w1s_ref[...], f, (((1,), (0,)), ((), ())),
            precision=lax.Precision.HIGHEST,
            preferred_element_type=jnp.float32)
        h = jnp.maximum(hp, 0.0)
        hf = jax.lax.dot_general(
            w2s_ref[...], h, (((1,), (0,)), ((), ())),
            precision=lax.Precision.HIGHEST,
            preferred_element_type=jnp.float32)
        alpha = 1.0 - jnp.exp(-jnp.exp(hf[0:1, :]) * STEP_LENGTH)

        dd = (d_ref[...] + 1.0) * 0.5 * 2.0 - 1.0
        x, y, z = dd[0:1, :], dd[1:2, :], dd[2:3, :]
        xy, xz, yz = x * y, x * z, y * z
        x2, y2, z2 = x * x, y * y, z * z
        sh = jnp.concatenate([
            jnp.full_like(x, 0.28209479177387814),
            -0.48860251190291987 * y,
            0.48860251190291987 * z,
            -0.48860251190291987 * x,
            1.0925484305920792 * xy,
            -1.0925484305920792 * yz,
            0.94617469575755997 * z2 - 0.31539156525251999,
            -1.0925484305920792 * xz,
            0.54627421529603959 * x2 - 0.54627421529603959 * y2,
            0.59004358992664352 * y * (-3.0 * x2 + y2),
            2.8906114426405538 * xy * z,
            0.45704579946446572 * y * (1.0 - 5.0 * z2),
            0.3731763325901154 * z * (5.0 * z2 - 3.0),
            0.45704579946446572 * x * (1.0 - 5.0 * z2),
            1.4453057213202769 * z * (x2 - y2),
            0.59004358992664352 * x * (x2 - 3.0 * y2),
        ], axis=0)

        feats2 = jnp.concatenate([hf, sh], axis=0)
        r = jnp.maximum(jax.lax.dot_general(
            w1r_ref[...], feats2, (((1,), (0,)), ((), ())),
            precision=lax.Precision.HIGHEST,
            preferred_element_type=jnp.float32), 0.0)
        r = jnp.maximum(jax.lax.dot_general(
            w2r_ref[...], r, (((1,), (0,)), ((), ())),
            precision=lax.Precision.HIGHEST,
            preferred_element_type=jnp.float32), 0.0)
        rgb = jax.nn.sigmoid(jax.lax.dot_general(
            w3r_ref[...], r, (((1,), (0,)), ((), ())),
            precision=lax.Precision.HIGHEST,
            preferred_element_type=jnp.float32))
        o_ref[...] = jnp.concatenate([rgb, alpha], axis=0)

    return pl.pallas_call(
        body,
        grid=(N // NB,),
        in_specs=[
            pl.BlockSpec((2 * N_LEVELS, NB), lambda i: (0, i)),
            pl.BlockSpec((3, NB), lambda i: (0, i)),
            pl.BlockSpec((64, 32), lambda i: (0, 0)),
            pl.BlockSpec((16, 64), lambda i: (0, 0)),
            pl.BlockSpec((64, 32), lambda i: (0, 0)),
            pl.BlockSpec((64, 64), lambda i: (0, 0)),
            pl.BlockSpec((3, 64), lambda i: (0, 0)),
        ],
        out_specs=pl.BlockSpec((4, NB), lambda i: (0, i)),
        out_shape=jax.ShapeDtypeStruct((4, N), jnp.float32),
    )(feats, dirT, w1s, w2s, w1r, w2r, w3r)


def kernel(position, direction, table, w_sig1, w_sig2, w_rgb1, w_rgb2,
           w_rgb3):
    px = position[:, 0]
    py = position[:, 1]
    pz = position[:, 2]
    tabf = table.reshape(N_LEVELS, T // 128, 128, F)
    tabf = tabf.transpose(0, 1, 3, 2).reshape(N_LEVELS * T * F)
    feats = _sc_hash_encode(px, py, pz, tabf)
    out4 = _tc_mlp(feats, direction.T, w_sig1.T, w_sig2.T, w_rgb1.T,
                   w_rgb2.T, w_rgb3.T)
    rgbs = out4[:3].T
    alphas = out4[3]
    return (rgbs, alphas)


# speedup vs baseline: 5.2678x; 1.5838x over previous
import math

import jax
import jax.numpy as jnp
import numpy as np
from jax import lax
from jax.experimental import pallas as pl
from jax.experimental.pallas import tpu as pltpu
from jax.experimental.pallas import tpu_sc as plsc

N_LEVELS = 16
F = 2
LOG2_T = 19
T = 1 << LOG2_T
MASK = T - 1
BASE_RES = 16
PER_LEVEL_SCALE = 1.5
P1 = int(np.uint32(2654435761).view(np.int32))
P2 = int(np.uint32(805459861).view(np.int32))
STEP_LENGTH = math.sqrt(3) / 1024

NC, NS = 2, 16
NW = NC * NS


def _res(l):
    return int(math.floor(BASE_RES * (PER_LEVEL_SCALE ** l)))


def _is_dense(l):
    return (_res(l) + 1) ** 3 <= T


_VLEVELS = (0, 1)
_VSIZE = tuple(((_res(l) + 1) ** 3 // 128 + 1) * 128 for l in _VLEVELS)


def _unpack2(g):
    f0 = plsc.bitcast(g << 16, jnp.float32)
    f1 = plsc.bitcast(g & jnp.int32(-65536), jnp.float32)
    return f0, f1


def _sc_hash_encode(px, py, pz, tabp):
    N = px.shape[0]
    npw = N // NW
    B = 512 if npw % 512 == 0 else npw
    G = B // 16
    NB128 = B // 128
    NF = 8 * NB128
    n_chunks = npw // B

    mesh = plsc.VectorSubcoreMesh(core_axis_name="c", subcore_axis_name="s",
                                  num_cores=NC, num_subcores=NS)

    def _coords(l, j, xb, yb, zb):
        res = _res(l)
        s = res + 1
        off = j * 16
        x = xb[pl.ds(off, 16)]
        y = yb[pl.ds(off, 16)]
        z = zb[pl.ds(off, 16)]
        posx = x * float(res)
        posy = y * float(res)
        posz = z * float(res)
        pix = posx.astype(jnp.int32)
        piy = posy.astype(jnp.int32)
        piz = posz.astype(jnp.int32)
        fx = posx - pix.astype(jnp.float32)
        fy = posy - piy.astype(jnp.float32)
        fz = posz - piz.astype(jnp.float32)
        if _is_dense(l):
            tx = (pix, pix + 1)
            ty = (piy * s, piy * s + s)
            tz = (piz * (s * s), piz * (s * s) + s * s)
        else:
            tx = (pix, pix + 1)
            ty = (piy * P1, piy * P1 + P1)
            tz = (piz * P2, piz * P2 + P2)
        wx = (1.0 - fx, fx)
        wy = (1.0 - fy, fy)
        wz = (1.0 - fz, fz)
        wxy = (wx[0] * wy[0], wx[1] * wy[0], wx[0] * wy[1], wx[1] * wy[1])
        cw = []
        for c in range(8):
            bx, bz = c & 1, (c >> 2) & 1
            if _is_dense(l):
                idx = tx[bx] + ty[(c >> 1) & 1] + tz[bz]
            else:
                idx = (tx[bx] ^ ty[(c >> 1) & 1] ^ tz[bz]) & MASK
            cw.append((idx, wxy[c & 3] * wz[bz]))
        return cw

    def body(px_h, py_h, pz_h, tab_h, out_h, xb, yb, zb, t0v, t1v,
             idxb0, idxb1, wb0, wb1, rowsb0, rowsb1, featb, sem):
        wid = lax.axis_index("s") * NC + lax.axis_index("c")
        pltpu.sync_copy(tab_h.at[pl.ds(0, _VSIZE[0])], t0v)
        pltpu.sync_copy(tab_h.at[pl.ds(T, _VSIZE[1])], t1v)
        bufs = ((idxb0, wb0, rowsb0), (idxb1, wb1, rowsb1))

        def fire(l, wait):
            idxb, _, rowsb = bufs[l & 1]

            def go(k, _):
                a = pltpu.make_async_copy(tab_h.at[idxb.at[k]],
                                          rowsb.at[k], sem)
                if wait:
                    a.wait()
                else:
                    a.start()
                return 0

            lax.fori_loop(0, NF, go, 0)

        def gen_level(l):
            idxb, wb, _ = bufs[l & 1]
            base = l * T

            def go(j, _):
                cw = _coords(l, j, xb, yb, zb)
                off = j * 16
                r8 = j // 8
                c8 = (j % 8) * 16
                for c, (idx, w) in enumerate(cw):
                    idxb[c * NB128 + r8, pl.ds(c8, 16)] = idx + base
                    wb[c, pl.ds(off, 16)] = w
                return 0

            lax.fori_loop(0, G, go, 0)

        def acc_level(l):
            _, wb, rowsb = bufs[l & 1]

            def go(j, _):
                off = j * 16
                r8 = j // 8
                c8 = (j % 8) * 16
                f0 = jnp.zeros((16,), jnp.float32)
                f1 = jnp.zeros((16,), jnp.float32)
                for c in range(8):
                    g = rowsb[c * NB128 + r8, pl.ds(c8, 16)]
                    w = wb[c, pl.ds(off, 16)]
                    g0, g1 = _unpack2(g)
                    f0 = f0 + w * g0
                    f1 = f1 + w * g1
                featb[2 * l, pl.ds(off, 16)] = f0
                featb[2 * l + 1, pl.ds(off, 16)] = f1
                return 0

            lax.fori_loop(0, G, go, 0)

        def vmem_level(l, tv):
            def go(j, _):
                cw = _coords(l, j, xb, yb, zb)
                off = j * 16
                f0 = jnp.zeros((16,), jnp.float32)
                f1 = jnp.zeros((16,), jnp.float32)
                for idx, w in cw:
                    g = plsc.load_gather(tv, [idx])
                    g0, g1 = _unpack2(g)
                    f0 = f0 + w * g0
                    f1 = f1 + w * g1
                featb[2 * l, pl.ds(off, 16)] = f0
                featb[2 * l + 1, pl.ds(off, 16)] = f1
                return 0

            lax.fori_loop(0, G, go, 0)

        def chunk_body(ci, _):
            base = wid * npw + ci * B
            pltpu.sync_copy(px_h.at[pl.ds(base, B)], xb)
            pltpu.sync_copy(py_h.at[pl.ds(base, B)], yb)
            pltpu.sync_copy(pz_h.at[pl.ds(base, B)], zb)

            gen_level(2)
            fire(2, False)
            vmem_level(0, t0v)
            vmem_level(1, t1v)
            for l in range(3, N_LEVELS):
                gen_level(l)
                fire(l - 1, True)
                fire(l, False)
                acc_level(l - 1)
            fire(N_LEVELS - 1, True)
            acc_level(N_LEVELS - 1)

            pltpu.sync_copy(featb, out_h.at[:, pl.ds(base, B)])
            return 0

        lax.fori_loop(0, n_chunks, chunk_body, 0)

    run = pl.kernel(
        body,
        out_type=jax.ShapeDtypeStruct((2 * N_LEVELS, N), jnp.float32),
        mesh=mesh,
        compiler_params=pltpu.CompilerParams(needs_layout_passes=False),
        scratch_types=[
            pltpu.VMEM((B,), jnp.float32),
            pltpu.VMEM((B,), jnp.float32),
            pltpu.VMEM((B,), jnp.float32),
            pltpu.VMEM((_VSIZE[0],), jnp.int32),
            pltpu.VMEM((_VSIZE[1],), jnp.int32),
            pltpu.VMEM((8 * (B // 128), 128), jnp.int32),
            pltpu.VMEM((8 * (B // 128), 128), jnp.int32),
            pltpu.VMEM((8, B), jnp.float32),
            pltpu.VMEM((8, B), jnp.float32),
            pltpu.VMEM((8 * (B // 128), 128), jnp.int32),
            pltpu.VMEM((8 * (B // 128), 128), jnp.int32),
            pltpu.VMEM((2 * N_LEVELS, B), jnp.float32),
            pltpu.SemaphoreType.DMA,
        ],
    )
    return run(px, py, pz, tabp)


def _tc_mlp(feats, dirT, w1s, w2s, w1r, w2r, w3r):
    N = feats.shape[1]
    NB = 2048 if N % 2048 == 0 else N

    def body(f_ref, d_ref, w1s_ref, w2s_ref, w1r_ref, w2r_ref, w3r_ref,
             o_ref):
        f = f_ref[...]
        hp = jax.lax.dot_general(
            w1s_ref[...], f, (((1,), (0,)), ((), ())),
            precision=lax.Precision.HIGHEST,
            preferred_element_type=jnp.float32)
        h = jnp.maximum(hp, 0.0)
        hf = jax.lax.dot_general(
            w2s_ref[...], h, (((1,), (0,)), ((), ())),
            precision=lax.Precision.HIGHEST,
            preferred_element_type=jnp.float32)
        alpha = 1.0 - jnp.exp(-jnp.exp(hf[0:1, :]) * STEP_LENGTH)

        dd = (d_ref[...] + 1.0) * 0.5 * 2.0 - 1.0
        x, y, z = dd[0:1, :], dd[1:2, :], dd[2:3, :]
        xy, xz, yz = x * y, x * z, y * z
        x2, y2, z2 = x * x, y * y, z * z
        sh = jnp.concatenate([
            jnp.full_like(x, 0.28209479177387814),
            -0.48860251190291987 * y,
            0.48860251190291987 * z,
            -0.48860251190291987 * x,
            1.0925484305920792 * xy,
            -1.0925484305920792 * yz,
            0.94617469575755997 * z2 - 0.31539156525251999,
            -1.0925484305920792 * xz,
            0.54627421529603959 * x2 - 0.54627421529603959 * y2,
            0.59004358992664352 * y * (-3.0 * x2 + y2),
            2.8906114426405538 * xy * z,
            0.45704579946446572 * y * (1.0 - 5.0 * z2),
            0.3731763325901154 * z * (5.0 * z2 - 3.0),
            0.45704579946446572 * x * (1.0 - 5.0 * z2),
            1.4453057213202769 * z * (x2 - y2),
            0.59004358992664352 * x * (x2 - 3.0 * y2),
        ], axis=0)

        feats2 = jnp.concatenate([hf, sh], axis=0)
        r = jnp.maximum(jax.lax.dot_general(
            w1r_ref[...], feats2, (((1,), (0,)), ((), ())),
            precision=lax.Precision.HIGHEST,
            preferred_element_type=jnp.float32), 0.0)
        r = jnp.maximum(jax.lax.dot_general(
            w2r_ref[...], r, (((1,), (0,)), ((), ())),
            precision=lax.Precision.HIGHEST,
            preferred_element_type=jnp.float32), 0.0)
        rgb = jax.nn.sigmoid(jax.lax.dot_general(
            w3r_ref[...], r, (((1,), (0,)), ((), ())),
            precision=lax.Precision.HIGHEST,
            preferred_element_type=jnp.float32))
        o_ref[...] = jnp.concatenate([rgb, alpha], axis=0)

    return pl.pallas_call(
        body,
        grid=(N // NB,),
        in_specs=[
            pl.BlockSpec((2 * N_LEVELS, NB), lambda i: (0, i)),
            pl.BlockSpec((3, NB), lambda i: (0, i)),
            pl.BlockSpec((64, 32), lambda i: (0, 0)),
            pl.BlockSpec((16, 64), lambda i: (0, 0)),
            pl.BlockSpec((64, 32), lambda i: (0, 0)),
            pl.BlockSpec((64, 64), lambda i: (0, 0)),
            pl.BlockSpec((3, 64), lambda i: (0, 0)),
        ],
        out_specs=pl.BlockSpec((4, NB), lambda i: (0, i)),
        out_shape=jax.ShapeDtypeStruct((4, N), jnp.float32),
    )(feats, dirT, w1s, w2s, w1r, w2r, w3r)


def kernel(position, direction, table, w_sig1, w_sig2, w_rgb1, w_rgb2,
           w_rgb3):
    px = position[:, 0]
    py = position[:, 1]
    pz = position[:, 2]
    t4 = table.reshape(N_LEVELS, T // 128, 128, F)
    b0 = lax.bitcast_convert_type(t4[..., 0].astype(jnp.bfloat16),
                                  jnp.uint16).astype(jnp.uint32)
    b1 = lax.bitcast_convert_type(t4[..., 1].astype(jnp.bfloat16),
                                  jnp.uint16).astype(jnp.uint32)
    tabp = lax.bitcast_convert_type(b0 | (b1 << 16),
                                    jnp.int32).reshape(N_LEVELS * T)
    feats = _sc_hash_encode(px, py, pz, tabp)
    out4 = _tc_mlp(feats, direction.T, w_sig1.T, w_sig2.T, w_rgb1.T,
                   w_rgb2.T, w_rgb3.T)
    rgbs = out4[:3].T
    alphas = out4[3]
    return (rgbs, alphas)


# speedup vs baseline: 5.6140x; 1.0657x over previous
import math

import jax
import jax.numpy as jnp
import numpy as np
from jax import lax
from jax.experimental import pallas as pl
from jax.experimental.pallas import tpu as pltpu
from jax.experimental.pallas import tpu_sc as plsc

N_LEVELS = 16
F = 2
LOG2_T = 19
T = 1 << LOG2_T
MASK = T - 1
BASE_RES = 16
PER_LEVEL_SCALE = 1.5
P1 = int(np.uint32(2654435761).view(np.int32))
P2 = int(np.uint32(805459861).view(np.int32))
STEP_LENGTH = math.sqrt(3) / 1024

NC, NS = 2, 16
NW = NC * NS


def _res(l):
    return int(math.floor(BASE_RES * (PER_LEVEL_SCALE ** l)))


def _is_dense(l):
    return (_res(l) + 1) ** 3 <= T


_VLEVELS = (0, 1, 2)
_VSIZE = tuple(((_res(l) + 1) ** 3 // 128 + 1) * 128 for l in _VLEVELS)


def _unpack2(g):
    f0 = plsc.bitcast(g << 16, jnp.float32)
    f1 = plsc.bitcast(g & jnp.int32(-65536), jnp.float32)
    return f0, f1


def _sc_hash_encode(px, py, pz, tabp):
    N = px.shape[0]
    npw = N // NW
    B = 512 if npw % 512 == 0 else npw
    G = B // 16
    NB128 = B // 128
    NF = 8 * NB128
    n_chunks = npw // B

    mesh = plsc.VectorSubcoreMesh(core_axis_name="c", subcore_axis_name="s",
                                  num_cores=NC, num_subcores=NS)

    def _coords(l, j, xb, yb, zb):
        res = _res(l)
        s = res + 1
        off = j * 16
        x = xb[pl.ds(off, 16)]
        y = yb[pl.ds(off, 16)]
        z = zb[pl.ds(off, 16)]
        posx = x * float(res)
        posy = y * float(res)
        posz = z * float(res)
        pix = posx.astype(jnp.int32)
        piy = posy.astype(jnp.int32)
        piz = posz.astype(jnp.int32)
        fx = posx - pix.astype(jnp.float32)
        fy = posy - piy.astype(jnp.float32)
        fz = posz - piz.astype(jnp.float32)
        if _is_dense(l):
            tx = (pix, pix + 1)
            ty = (piy * s, piy * s + s)
            tz = (piz * (s * s), piz * (s * s) + s * s)
        else:
            tx = (pix, pix + 1)
            ty = (piy * P1, piy * P1 + P1)
            tz = (piz * P2, piz * P2 + P2)
        wx = (1.0 - fx, fx)
        wy = (1.0 - fy, fy)
        wz = (1.0 - fz, fz)
        wxy = (wx[0] * wy[0], wx[1] * wy[0], wx[0] * wy[1], wx[1] * wy[1])
        cw = []
        for c in range(8):
            bx, bz = c & 1, (c >> 2) & 1
            if _is_dense(l):
                idx = tx[bx] + ty[(c >> 1) & 1] + tz[bz]
            else:
                idx = (tx[bx] ^ ty[(c >> 1) & 1] ^ tz[bz]) & MASK
            cw.append((idx, wxy[c & 3] * wz[bz]))
        return cw

    def body(px_h, py_h, pz_h, tab_h, out_h, xb, yb, zb, t0v, t1v, t2v,
             idxb0, idxb1, wb0, wb1, rowsb0, rowsb1, featb, sem):
        wid = lax.axis_index("s") * NC + lax.axis_index("c")
        pltpu.sync_copy(tab_h.at[pl.ds(0, _VSIZE[0])], t0v)
        pltpu.sync_copy(tab_h.at[pl.ds(T, _VSIZE[1])], t1v)
        pltpu.sync_copy(tab_h.at[pl.ds(2 * T, _VSIZE[2])], t2v)
        bufs = ((idxb0, wb0, rowsb0), (idxb1, wb1, rowsb1))

        def fire(l, wait):
            idxb, _, rowsb = bufs[l & 1]

            def go(k, _):
                a = pltpu.make_async_copy(tab_h.at[idxb.at[k]],
                                          rowsb.at[k], sem)
                if wait:
                    a.wait()
                else:
                    a.start()
                return 0

            lax.fori_loop(0, NF, go, 0)

        def gen_level(l):
            idxb, wb, _ = bufs[l & 1]
            base = l * T

            def go(j, _):
                cw = _coords(l, j, xb, yb, zb)
                off = j * 16
                r8 = j // 8
                c8 = (j % 8) * 16
                for c, (idx, w) in enumerate(cw):
                    idxb[c * NB128 + r8, pl.ds(c8, 16)] = idx + base
                    wb[c, pl.ds(off, 16)] = w
                return 0

            lax.fori_loop(0, G, go, 0)

        def acc_level(l):
            _, wb, rowsb = bufs[l & 1]

            def go(j, _):
                off = j * 16
                r8 = j // 8
                c8 = (j % 8) * 16
                f0 = jnp.zeros((16,), jnp.float32)
                f1 = jnp.zeros((16,), jnp.float32)
                for c in range(8):
                    g = rowsb[c * NB128 + r8, pl.ds(c8, 16)]
                    w = wb[c, pl.ds(off, 16)]
                    g0, g1 = _unpack2(g)
                    f0 = f0 + w * g0
                    f1 = f1 + w * g1
                featb[2 * l, pl.ds(off, 16)] = f0
                featb[2 * l + 1, pl.ds(off, 16)] = f1
                return 0

            lax.fori_loop(0, G, go, 0)

        def vmem_level(l, tv):
            def go(j, _):
                cw = _coords(l, j, xb, yb, zb)
                off = j * 16
                f0 = jnp.zeros((16,), jnp.float32)
                f1 = jnp.zeros((16,), jnp.float32)
                for idx, w in cw:
                    g = plsc.load_gather(tv, [idx])
                    g0, g1 = _unpack2(g)
                    f0 = f0 + w * g0
                    f1 = f1 + w * g1
                featb[2 * l, pl.ds(off, 16)] = f0
                featb[2 * l + 1, pl.ds(off, 16)] = f1
                return 0

            lax.fori_loop(0, G, go, 0)

        def chunk_body(ci, _):
            base = wid * npw + ci * B
            pltpu.sync_copy(px_h.at[pl.ds(base, B)], xb)
            pltpu.sync_copy(py_h.at[pl.ds(base, B)], yb)
            pltpu.sync_copy(pz_h.at[pl.ds(base, B)], zb)

            gen_level(3)
            fire(3, False)
            vmem_level(0, t0v)
            vmem_level(1, t1v)
            vmem_level(2, t2v)
            for l in range(4, N_LEVELS):
                gen_level(l)
                fire(l - 1, True)
                fire(l, False)
                acc_level(l - 1)
            fire(N_LEVELS - 1, True)
            acc_level(N_LEVELS - 1)

            pltpu.sync_copy(featb, out_h.at[:, pl.ds(base, B)])
            return 0

        lax.fori_loop(0, n_chunks, chunk_body, 0)

    run = pl.kernel(
        body,
        out_type=jax.ShapeDtypeStruct((2 * N_LEVELS, N), jnp.float32),
        mesh=mesh,
        compiler_params=pltpu.CompilerParams(needs_layout_passes=False),
        scratch_types=[
            pltpu.VMEM((B,), jnp.float32),
            pltpu.VMEM((B,), jnp.float32),
            pltpu.VMEM((B,), jnp.float32),
            pltpu.VMEM((_VSIZE[0],), jnp.int32),
            pltpu.VMEM((_VSIZE[1],), jnp.int32),
            pltpu.VMEM((_VSIZE[2],), jnp.int32),
            pltpu.VMEM((8 * (B // 128), 128), jnp.int32),
            pltpu.VMEM((8 * (B // 128), 128), jnp.int32),
            pltpu.VMEM((8, B), jnp.float32),
            pltpu.VMEM((8, B), jnp.float32),
            pltpu.VMEM((8 * (B // 128), 128), jnp.int32),
            pltpu.VMEM((8 * (B // 128), 128), jnp.int32),
            pltpu.VMEM((2 * N_LEVELS, B), jnp.float32),
            pltpu.SemaphoreType.DMA,
        ],
    )
    return run(px, py, pz, tabp)


def _tc_mlp(feats, dirT, w1s, w2s, w1r, w2r, w3r):
    N = feats.shape[1]
    NB = 2048 if N % 2048 == 0 else N

    def body(f_ref, d_ref, w1s_ref, w2s_ref, w1r_ref, w2r_ref, w3r_ref,
             o_ref):
        f = f_ref[...]
        hp = jax.lax.dot_general(
            w1s_ref[...], f, (((1,), (0,)), ((), ())),
            precision=lax.Precision.HIGHEST,
            preferred_element_type=jnp.float32)
        h = jnp.maximum(hp, 0.0)
        hf = jax.lax.dot_general(
            w2s_ref[...], h, (((1,), (0,)), ((), ())),
            precision=lax.Precision.HIGHEST,
            preferred_element_type=jnp.float32)
        alpha = 1.0 - jnp.exp(-jnp.exp(hf[0:1, :]) * STEP_LENGTH)

        dd = (d_ref[...] + 1.0) * 0.5 * 2.0 - 1.0
        x, y, z = dd[0:1, :], dd[1:2, :], dd[2:3, :]
        xy, xz, yz = x * y, x * z, y * z
        x2, y2, z2 = x * x, y * y, z * z
        sh = jnp.concatenate([
            jnp.full_like(x, 0.28209479177387814),
            -0.48860251190291987 * y,
            0.48860251190291987 * z,
            -0.48860251190291987 * x,
            1.0925484305920792 * xy,
            -1.0925484305920792 * yz,
            0.94617469575755997 * z2 - 0.31539156525251999,
            -1.0925484305920792 * xz,
            0.54627421529603959 * x2 - 0.54627421529603959 * y2,
            0.59004358992664352 * y * (-3.0 * x2 + y2),
            2.8906114426405538 * xy * z,
            0.45704579946446572 * y * (1.0 - 5.0 * z2),
            0.3731763325901154 * z * (5.0 * z2 - 3.0),
            0.45704579946446572 * x * (1.0 - 5.0 * z2),
            1.4453057213202769 * z * (x2 - y2),
            0.59004358992664352 * x * (x2 - 3.0 * y2),
        ], axis=0)

        feats2 = jnp.concatenate([hf, sh], axis=0)
        r = jnp.maximum(jax.lax.dot_general(
            w1r_ref[...], feats2, (((1,), (0,)), ((), ())),
            precision=lax.Precision.HIGHEST,
            preferred_element_type=jnp.float32), 0.0)
        r = jnp.maximum(jax.lax.dot_general(
            w2r_ref[...], r, (((1,), (0,)), ((), ())),
            precision=lax.Precision.HIGHEST,
            preferred_element_type=jnp.float32), 0.0)
        rgb = jax.nn.sigmoid(jax.lax.dot_general(
            w3r_ref[...], r, (((1,), (0,)), ((), ())),
            precision=lax.Precision.HIGHEST,
            preferred_element_type=jnp.float32))
        o_ref[...] = jnp.concatenate([rgb, alpha], axis=0)

    return pl.pallas_call(
        body,
        grid=(N // NB,),
        in_specs=[
            pl.BlockSpec((2 * N_LEVELS, NB), lambda i: (0, i)),
            pl.BlockSpec((3, NB), lambda i: (0, i)),
            pl.BlockSpec((64, 32), lambda i: (0, 0)),
            pl.BlockSpec((16, 64), lambda i: (0, 0)),
            pl.BlockSpec((64, 32), lambda i: (0, 0)),
            pl.BlockSpec((64, 64), lambda i: (0, 0)),
            pl.BlockSpec((3, 64), lambda i: (0, 0)),
        ],
        out_specs=pl.BlockSpec((4, NB), lambda i: (0, i)),
        out_shape=jax.ShapeDtypeStruct((4, N), jnp.float32),
    )(feats, dirT, w1s, w2s, w1r, w2r, w3r)


def kernel(position, direction, table, w_sig1, w_sig2, w_rgb1, w_rgb2,
           w_rgb3):
    px = position[:, 0]
    py = position[:, 1]
    pz = position[:, 2]
    t4 = table.reshape(N_LEVELS, T // 128, 128, F)
    b0 = lax.bitcast_convert_type(t4[..., 0].astype(jnp.bfloat16),
                                  jnp.uint16).astype(jnp.uint32)
    b1 = lax.bitcast_convert_type(t4[..., 1].astype(jnp.bfloat16),
                                  jnp.uint16).astype(jnp.uint32)
    tabp = lax.bitcast_convert_type(b0 | (b1 << 16),
                                    jnp.int32).reshape(N_LEVELS * T)
    feats = _sc_hash_encode(px, py, pz, tabp)
    out4 = _tc_mlp(feats, direction.T, w_sig1.T, w_sig2.T, w_rgb1.T,
                   w_rgb2.T, w_rgb3.T)
    rgbs = out4[:3].T
    alphas = out4[3]
    return (rgbs, alphas)


# speedup vs baseline: 6.3475x; 1.1306x over previous
import math

import jax
import jax.numpy as jnp
import numpy as np
from jax import lax
from jax.experimental import pallas as pl
from jax.experimental.pallas import tpu as pltpu
from jax.experimental.pallas import tpu_sc as plsc

N_LEVELS = 16
F = 2
LOG2_T = 19
T = 1 << LOG2_T
MASK = T - 1
BASE_RES = 16
PER_LEVEL_SCALE = 1.5
P1 = int(np.uint32(2654435761).view(np.int32))
P2 = int(np.uint32(805459861).view(np.int32))
STEP_LENGTH = math.sqrt(3) / 1024

NC, NS = 2, 16
NW = NC * NS


def _res(l):
    return int(math.floor(BASE_RES * (PER_LEVEL_SCALE ** l)))


def _is_dense(l):
    return (_res(l) + 1) ** 3 <= T


_VLEVELS = (0, 1, 2)
_VSIZE = tuple(((_res(l) + 1) ** 3 // 128 + 1) * 128 for l in _VLEVELS)


def _unpack2(g):
    f0 = plsc.bitcast(g << 16, jnp.float32)
    f1 = plsc.bitcast(g & jnp.int32(-65536), jnp.float32)
    return f0, f1


def _sc_hash_encode(px, py, pz, tabp):
    N = px.shape[0]
    npw = N // NW
    B = 512 if npw % 512 == 0 else npw
    G = B // 16
    NB128 = B // 128
    NF = 8 * NB128
    n_chunks = npw // B

    mesh = plsc.VectorSubcoreMesh(core_axis_name="c", subcore_axis_name="s",
                                  num_cores=NC, num_subcores=NS)

    def _coords(l, j, xb, yb, zb):
        res = _res(l)
        s = res + 1
        off = j * 16
        x = xb[pl.ds(off, 16)]
        y = yb[pl.ds(off, 16)]
        z = zb[pl.ds(off, 16)]
        posx = x * float(res)
        posy = y * float(res)
        posz = z * float(res)
        pix = posx.astype(jnp.int32)
        piy = posy.astype(jnp.int32)
        piz = posz.astype(jnp.int32)
        fx = posx - pix.astype(jnp.float32)
        fy = posy - piy.astype(jnp.float32)
        fz = posz - piz.astype(jnp.float32)
        if _is_dense(l):
            tx = (pix, pix + 1)
            ty = (piy * s, piy * s + s)
            tz = (piz * (s * s), piz * (s * s) + s * s)
        else:
            tx = (pix, pix + 1)
            ty = (piy * P1, piy * P1 + P1)
            tz = (piz * P2, piz * P2 + P2)
        wx = (1.0 - fx, fx)
        wy = (1.0 - fy, fy)
        wz = (1.0 - fz, fz)
        wxy = (wx[0] * wy[0], wx[1] * wy[0], wx[0] * wy[1], wx[1] * wy[1])
        cw = []
        for c in range(8):
            bx, bz = c & 1, (c >> 2) & 1
            if _is_dense(l):
                idx = tx[bx] + ty[(c >> 1) & 1] + tz[bz]
            else:
                idx = (tx[bx] ^ ty[(c >> 1) & 1] ^ tz[bz]) & MASK
            cw.append((idx, wxy[c & 3] * wz[bz]))
        return cw

    def body(px_h, py_h, pz_h, tab_h, out_h, xb, yb, zb, t0v, t1v, t2v,
             idxb0, idxb1, wb0, wb1, rowsb0, rowsb1, featb, sem):
        wid = lax.axis_index("s") * NC + lax.axis_index("c")
        pltpu.sync_copy(tab_h.at[pl.ds(0, _VSIZE[0])], t0v)
        pltpu.sync_copy(tab_h.at[pl.ds(T, _VSIZE[1])], t1v)
        pltpu.sync_copy(tab_h.at[pl.ds(2 * T, _VSIZE[2])], t2v)
        bufs = ((idxb0, wb0, rowsb0), (idxb1, wb1, rowsb1))

        def fire(l, wait):
            idxb, _, rowsb = bufs[l & 1]

            def go(k, _):
                a = pltpu.make_async_copy(tab_h.at[idxb.at[k]],
                                          rowsb.at[k], sem)
                if wait:
                    a.wait()
                else:
                    a.start()
                return 0

            lax.fori_loop(0, NF, go, 0)

        def gen_level(l):
            idxb, wb, _ = bufs[l & 1]
            base = l * T

            def go(j, _):
                cw = _coords(l, j, xb, yb, zb)
                off = j * 16
                r8 = j // 8
                c8 = (j % 8) * 16
                for c, (idx, w) in enumerate(cw):
                    idxb[c * NB128 + r8, pl.ds(c8, 16)] = idx + base
                    wb[c, pl.ds(off, 16)] = w
                return 0

            lax.fori_loop(0, G, go, 0)

        def acc_level(l):
            _, wb, rowsb = bufs[l & 1]

            def go(j, _):
                off = j * 16
                r8 = j // 8
                c8 = (j % 8) * 16
                f0 = jnp.zeros((16,), jnp.float32)
                f1 = jnp.zeros((16,), jnp.float32)
                for c in range(8):
                    g = rowsb[c * NB128 + r8, pl.ds(c8, 16)]
                    w = wb[c, pl.ds(off, 16)]
                    g0, g1 = _unpack2(g)
                    f0 = f0 + w * g0
                    f1 = f1 + w * g1
                featb[2 * l, pl.ds(off, 16)] = f0
                featb[2 * l + 1, pl.ds(off, 16)] = f1
                return 0

            lax.fori_loop(0, G, go, 0)

        def vmem_level(l, tv):
            def go(j, _):
                cw = _coords(l, j, xb, yb, zb)
                off = j * 16
                f0 = jnp.zeros((16,), jnp.float32)
                f1 = jnp.zeros((16,), jnp.float32)
                for idx, w in cw:
                    g = plsc.load_gather(tv, [idx])
                    g0, g1 = _unpack2(g)
                    f0 = f0 + w * g0
                    f1 = f1 + w * g1
                featb[2 * l, pl.ds(off, 16)] = f0
                featb[2 * l + 1, pl.ds(off, 16)] = f1
                return 0

            lax.fori_loop(0, G, go, 0)

        def chunk_body(ci, _):
            base = wid * npw + ci * B
            pltpu.sync_copy(px_h.at[pl.ds(base, B)], xb)
            pltpu.sync_copy(py_h.at[pl.ds(base, B)], yb)
            pltpu.sync_copy(pz_h.at[pl.ds(base, B)], zb)

            gen_level(3)
            fire(3, False)
            vmem_level(0, t0v)
            vmem_level(1, t1v)
            vmem_level(2, t2v)
            for l in range(4, N_LEVELS):
                gen_level(l)
                fire(l - 1, True)
                fire(l, False)
                acc_level(l - 1)
            fire(N_LEVELS - 1, True)
            acc_level(N_LEVELS - 1)

            pltpu.sync_copy(featb, out_h.at[:, pl.ds(base, B)])
            return 0

        lax.fori_loop(0, n_chunks, chunk_body, 0)

    run = pl.kernel(
        body,
        out_type=jax.ShapeDtypeStruct((2 * N_LEVELS, N), jnp.float32),
        mesh=mesh,
        compiler_params=pltpu.CompilerParams(needs_layout_passes=False),
        scratch_types=[
            pltpu.VMEM((B,), jnp.float32),
            pltpu.VMEM((B,), jnp.float32),
            pltpu.VMEM((B,), jnp.float32),
            pltpu.VMEM((_VSIZE[0],), jnp.int32),
            pltpu.VMEM((_VSIZE[1],), jnp.int32),
            pltpu.VMEM((_VSIZE[2],), jnp.int32),
            pltpu.VMEM((8 * (B // 128), 128), jnp.int32),
            pltpu.VMEM((8 * (B // 128), 128), jnp.int32),
            pltpu.VMEM((8, B), jnp.float32),
            pltpu.VMEM((8, B), jnp.float32),
            pltpu.VMEM((8 * (B // 128), 128), jnp.int32),
            pltpu.VMEM((8 * (B // 128), 128), jnp.int32),
            pltpu.VMEM((2 * N_LEVELS, B), jnp.float32),
            pltpu.SemaphoreType.DMA,
        ],
    )
    return run(px, py, pz, tabp)


def _tc_mlp(feats, dirT, w1s, w2s, w1r, w2r, w3r):
    N = feats.shape[1]
    NB = 4096 if N % 4096 == 0 else N

    def body(f_ref, d_ref, w1s_ref, w2s_ref, w1r_ref, w2r_ref, w3r_ref,
             o_ref):
        f = f_ref[...]
        hp = jax.lax.dot_general(
            w1s_ref[...], f, (((1,), (0,)), ((), ())),
            preferred_element_type=jnp.float32)
        h = jnp.maximum(hp, 0.0)
        hf = jax.lax.dot_general(
            w2s_ref[...], h, (((1,), (0,)), ((), ())),
            preferred_element_type=jnp.float32)
        alpha = 1.0 - jnp.exp(-jnp.exp(hf[0:1, :]) * STEP_LENGTH)

        dd = (d_ref[...] + 1.0) * 0.5 * 2.0 - 1.0
        x, y, z = dd[0:1, :], dd[1:2, :], dd[2:3, :]
        xy, xz, yz = x * y, x * z, y * z
        x2, y2, z2 = x * x, y * y, z * z
        sh = jnp.concatenate([
            jnp.full_like(x, 0.28209479177387814),
            -0.48860251190291987 * y,
            0.48860251190291987 * z,
            -0.48860251190291987 * x,
            1.0925484305920792 * xy,
            -1.0925484305920792 * yz,
            0.94617469575755997 * z2 - 0.31539156525251999,
            -1.0925484305920792 * xz,
            0.54627421529603959 * x2 - 0.54627421529603959 * y2,
            0.59004358992664352 * y * (-3.0 * x2 + y2),
            2.8906114426405538 * xy * z,
            0.45704579946446572 * y * (1.0 - 5.0 * z2),
            0.3731763325901154 * z * (5.0 * z2 - 3.0),
            0.45704579946446572 * x * (1.0 - 5.0 * z2),
            1.4453057213202769 * z * (x2 - y2),
            0.59004358992664352 * x * (x2 - 3.0 * y2),
        ], axis=0)

        feats2 = jnp.concatenate([hf, sh], axis=0)
        r = jnp.maximum(jax.lax.dot_general(
            w1r_ref[...], feats2, (((1,), (0,)), ((), ())),
            preferred_element_type=jnp.float32), 0.0)
        r = jnp.maximum(jax.lax.dot_general(
            w2r_ref[...], r, (((1,), (0,)), ((), ())),
            preferred_element_type=jnp.float32), 0.0)
        rgb = jax.nn.sigmoid(jax.lax.dot_general(
            w3r_ref[...], r, (((1,), (0,)), ((), ())),
            preferred_element_type=jnp.float32))
        o_ref[...] = jnp.concatenate([rgb, alpha], axis=0)

    return pl.pallas_call(
        body,
        grid=(N // NB,),
        in_specs=[
            pl.BlockSpec((2 * N_LEVELS, NB), lambda i: (0, i)),
            pl.BlockSpec((3, NB), lambda i: (0, i)),
            pl.BlockSpec((64, 32), lambda i: (0, 0)),
            pl.BlockSpec((16, 64), lambda i: (0, 0)),
            pl.BlockSpec((64, 32), lambda i: (0, 0)),
            pl.BlockSpec((64, 64), lambda i: (0, 0)),
            pl.BlockSpec((3, 64), lambda i: (0, 0)),
        ],
        out_specs=pl.BlockSpec((4, NB), lambda i: (0, i)),
        out_shape=jax.ShapeDtypeStruct((4, N), jnp.float32),
    )(feats, dirT, w1s, w2s, w1r, w2r, w3r)


def kernel(position, direction, table, w_sig1, w_sig2, w_rgb1, w_rgb2,
           w_rgb3):
    px = position[:, 0]
    py = position[:, 1]
    pz = position[:, 2]
    t4 = table.reshape(N_LEVELS, T // 128, 128, F)
    b0 = lax.bitcast_convert_type(t4[..., 0].astype(jnp.bfloat16),
                                  jnp.uint16).astype(jnp.uint32)
    b1 = lax.bitcast_convert_type(t4[..., 1].astype(jnp.bfloat16),
                                  jnp.uint16).astype(jnp.uint32)
    tabp = lax.bitcast_convert_type(b0 | (b1 << 16),
                                    jnp.int32).reshape(N_LEVELS * T)
    feats = _sc_hash_encode(px, py, pz, tabp)
    out4 = _tc_mlp(feats, direction.T, w_sig1.T, w_sig2.T, w_rgb1.T,
                   w_rgb2.T, w_rgb3.T)
    rgbs = out4[:3].T
    alphas = out4[3]
    return (rgbs, alphas)


# speedup vs baseline: 6.4342x; 1.0137x over previous
import math

import jax
import jax.numpy as jnp
import numpy as np
from jax import lax
from jax.experimental import pallas as pl
from jax.experimental.pallas import tpu as pltpu
from jax.experimental.pallas import tpu_sc as plsc

N_LEVELS = 16
F = 2
LOG2_T = 19
T = 1 << LOG2_T
MASK = T - 1
BASE_RES = 16
PER_LEVEL_SCALE = 1.5
P1 = int(np.uint32(2654435761).view(np.int32))
P2 = int(np.uint32(805459861).view(np.int32))
STEP_LENGTH = math.sqrt(3) / 1024

NC, NS = 2, 16
NW = NC * NS


def _res(l):
    return int(math.floor(BASE_RES * (PER_LEVEL_SCALE ** l)))


def _is_dense(l):
    return (_res(l) + 1) ** 3 <= T


_VLEVELS = (0, 1, 2)
_VSIZE = tuple(((_res(l) + 1) ** 3 // 128 + 1) * 128 for l in _VLEVELS)


def _unpack2(g):
    f0 = plsc.bitcast(g << 16, jnp.float32)
    f1 = plsc.bitcast(g & jnp.int32(-65536), jnp.float32)
    return f0, f1


def _sc_hash_encode(px, py, pz, tabp):
    N = px.shape[0]
    npw = N // NW
    B = 512 if npw % 512 == 0 else npw
    G = B // 16
    NB128 = B // 128
    NF = 8 * NB128
    n_chunks = npw // B

    mesh = plsc.VectorSubcoreMesh(core_axis_name="c", subcore_axis_name="s",
                                  num_cores=NC, num_subcores=NS)

    def _coords(l, j, xb, yb, zb):
        res = _res(l)
        s = res + 1
        off = j * 16
        x = xb[pl.ds(off, 16)]
        y = yb[pl.ds(off, 16)]
        z = zb[pl.ds(off, 16)]
        posx = x * float(res)
        posy = y * float(res)
        posz = z * float(res)
        pix = posx.astype(jnp.int32)
        piy = posy.astype(jnp.int32)
        piz = posz.astype(jnp.int32)
        fx = posx - pix.astype(jnp.float32)
        fy = posy - piy.astype(jnp.float32)
        fz = posz - piz.astype(jnp.float32)
        if _is_dense(l):
            tx = (pix, pix + 1)
            ty = (piy * s, piy * s + s)
            tz = (piz * (s * s), piz * (s * s) + s * s)
        else:
            tx = (pix, pix + 1)
            ty = (piy * P1, piy * P1 + P1)
            tz = (piz * P2, piz * P2 + P2)
        wx = (1.0 - fx, fx)
        wy = (1.0 - fy, fy)
        wz = (1.0 - fz, fz)
        wxy = (wx[0] * wy[0], wx[1] * wy[0], wx[0] * wy[1], wx[1] * wy[1])
        cw = []
        for c in range(8):
            bx, bz = c & 1, (c >> 2) & 1
            if _is_dense(l):
                idx = tx[bx] + ty[(c >> 1) & 1] + tz[bz]
            else:
                idx = (tx[bx] ^ ty[(c >> 1) & 1] ^ tz[bz]) & MASK
            cw.append((idx, wxy[c & 3] * wz[bz]))
        return cw

    def body(px_h, py_h, pz_h, tab_h, out_h, xb, yb, zb, t0v, t1v, t2v,
             idxb0, idxb1, wb0, wb1, rowsb0, rowsb1, featb, sem):
        wid = lax.axis_index("s") * NC + lax.axis_index("c")
        pltpu.sync_copy(tab_h.at[pl.ds(0, _VSIZE[0])], t0v)
        pltpu.sync_copy(tab_h.at[pl.ds(T, _VSIZE[1])], t1v)
        pltpu.sync_copy(tab_h.at[pl.ds(2 * T, _VSIZE[2])], t2v)
        bufs = ((idxb0, wb0, rowsb0), (idxb1, wb1, rowsb1))

        def fire(l, wait):
            idxb, _, rowsb = bufs[l & 1]

            def go(k, _):
                a = pltpu.make_async_copy(tab_h.at[idxb.at[k]],
                                          rowsb.at[k], sem)
                if wait:
                    a.wait()
                else:
                    a.start()
                return 0

            lax.fori_loop(0, NF, go, 0)

        def gen_level(l):
            idxb, wb, _ = bufs[l & 1]
            base = l * T

            def go(j, _):
                cw = _coords(l, j, xb, yb, zb)
                off = j * 16
                r8 = j // 8
                c8 = (j % 8) * 16
                for c, (idx, w) in enumerate(cw):
                    idxb[c * NB128 + r8, pl.ds(c8, 16)] = idx + base
                    wb[c, pl.ds(off, 16)] = w
                return 0

            lax.fori_loop(0, G, go, 0)

        def acc_level(l):
            _, wb, rowsb = bufs[l & 1]

            def go(j, _):
                off = j * 16
                r8 = j // 8
                c8 = (j % 8) * 16
                f0 = jnp.zeros((16,), jnp.float32)
                f1 = jnp.zeros((16,), jnp.float32)
                for c in range(8):
                    g = rowsb[c * NB128 + r8, pl.ds(c8, 16)]
                    w = wb[c, pl.ds(off, 16)]
                    g0, g1 = _unpack2(g)
                    f0 = f0 + w * g0
                    f1 = f1 + w * g1
                featb[2 * l, pl.ds(off, 16)] = f0
                featb[2 * l + 1, pl.ds(off, 16)] = f1
                return 0

            lax.fori_loop(0, G, go, 0)

        def vmem_level(l, tv):
            def go(j, _):
                cw = _coords(l, j, xb, yb, zb)
                off = j * 16
                f0 = jnp.zeros((16,), jnp.float32)
                f1 = jnp.zeros((16,), jnp.float32)
                for idx, w in cw:
                    g = plsc.load_gather(tv, [idx])
                    g0, g1 = _unpack2(g)
                    f0 = f0 + w * g0
                    f1 = f1 + w * g1
                featb[2 * l, pl.ds(off, 16)] = f0
                featb[2 * l + 1, pl.ds(off, 16)] = f1
                return 0

            lax.fori_loop(0, G, go, 0)

        def chunk_body(ci, _):
            base = wid * npw + ci * B
            pltpu.sync_copy(px_h.at[pl.ds(base, B)], xb)
            pltpu.sync_copy(py_h.at[pl.ds(base, B)], yb)
            pltpu.sync_copy(pz_h.at[pl.ds(base, B)], zb)

            gen_level(3)
            fire(3, False)
            vmem_level(0, t0v)
            vmem_level(1, t1v)
            vmem_level(2, t2v)
            for l in range(4, N_LEVELS):
                gen_level(l)
                fire(l - 1, True)
                fire(l, False)
                acc_level(l - 1)
            fire(N_LEVELS - 1, True)
            acc_level(N_LEVELS - 1)

            pltpu.sync_copy(featb, out_h.at[:, pl.ds(base, B)])
            return 0

        lax.fori_loop(0, n_chunks, chunk_body, 0)

    run = pl.kernel(
        body,
        out_type=jax.ShapeDtypeStruct((2 * N_LEVELS, N), jnp.float32),
        mesh=mesh,
        compiler_params=pltpu.CompilerParams(needs_layout_passes=False),
        scratch_types=[
            pltpu.VMEM((B,), jnp.float32),
            pltpu.VMEM((B,), jnp.float32),
            pltpu.VMEM((B,), jnp.float32),
            pltpu.VMEM((_VSIZE[0],), jnp.int32),
            pltpu.VMEM((_VSIZE[1],), jnp.int32),
            pltpu.VMEM((_VSIZE[2],), jnp.int32),
            pltpu.VMEM((8 * (B // 128), 128), jnp.int32),
            pltpu.VMEM((8 * (B // 128), 128), jnp.int32),
            pltpu.VMEM((8, B), jnp.float32),
            pltpu.VMEM((8, B), jnp.float32),
            pltpu.VMEM((8 * (B // 128), 128), jnp.int32),
            pltpu.VMEM((8 * (B // 128), 128), jnp.int32),
            pltpu.VMEM((2 * N_LEVELS, B), jnp.float32),
            pltpu.SemaphoreType.DMA,
        ],
    )
    return run(px, py, pz, tabp)


def _tc_mlp(feats, dirT, w1s, w2s, w1r, w2r, w3r):
    N = feats.shape[1]
    NB = 4096 if N % 4096 == 0 else N

    def body(f_ref, d_ref, w1s_ref, w2s_ref, w1r_ref, w2r_ref, w3r_ref,
             o_ref):
        f = f_ref[...]
        hp = jax.lax.dot_general(
            w1s_ref[...], f, (((1,), (0,)), ((), ())),
            preferred_element_type=jnp.float32)
        h = jnp.maximum(hp, 0.0)
        hf = jax.lax.dot_general(
            w2s_ref[...], h, (((1,), (0,)), ((), ())),
            preferred_element_type=jnp.float32)
        alpha = 1.0 - jnp.exp(-jnp.exp(hf[0:1, :]) * STEP_LENGTH)

        dd = (d_ref[...] + 1.0) * 0.5 * 2.0 - 1.0
        x, y, z = dd[0:1, :], dd[1:2, :], dd[2:3, :]
        xy, xz, yz = x * y, x * z, y * z
        x2, y2, z2 = x * x, y * y, z * z
        sh = jnp.concatenate([
            jnp.full_like(x, 0.28209479177387814),
            -0.48860251190291987 * y,
            0.48860251190291987 * z,
            -0.48860251190291987 * x,
            1.0925484305920792 * xy,
            -1.0925484305920792 * yz,
            0.94617469575755997 * z2 - 0.31539156525251999,
            -1.0925484305920792 * xz,
            0.54627421529603959 * x2 - 0.54627421529603959 * y2,
            0.59004358992664352 * y * (-3.0 * x2 + y2),
            2.8906114426405538 * xy * z,
            0.45704579946446572 * y * (1.0 - 5.0 * z2),
            0.3731763325901154 * z * (5.0 * z2 - 3.0),
            0.45704579946446572 * x * (1.0 - 5.0 * z2),
            1.4453057213202769 * z * (x2 - y2),
            0.59004358992664352 * x * (x2 - 3.0 * y2),
        ], axis=0)

        feats2 = jnp.concatenate([hf, sh], axis=0)
        r = jnp.maximum(jax.lax.dot_general(
            w1r_ref[...], feats2, (((1,), (0,)), ((), ())),
            preferred_element_type=jnp.float32), 0.0)
        r = jnp.maximum(jax.lax.dot_general(
            w2r_ref[...], r, (((1,), (0,)), ((), ())),
            preferred_element_type=jnp.float32), 0.0)
        rgb = jax.nn.sigmoid(jax.lax.dot_general(
            w3r_ref[...], r, (((1,), (0,)), ((), ())),
            preferred_element_type=jnp.float32))
        o_ref[...] = jnp.concatenate([rgb, alpha], axis=0)

    return pl.pallas_call(
        body,
        grid=(N // NB,),
        in_specs=[
            pl.BlockSpec((2 * N_LEVELS, NB), lambda i: (0, i)),
            pl.BlockSpec((3, NB), lambda i: (0, i)),
            pl.BlockSpec((64, 32), lambda i: (0, 0)),
            pl.BlockSpec((16, 64), lambda i: (0, 0)),
            pl.BlockSpec((64, 32), lambda i: (0, 0)),
            pl.BlockSpec((64, 64), lambda i: (0, 0)),
            pl.BlockSpec((3, 64), lambda i: (0, 0)),
        ],
        out_specs=pl.BlockSpec((4, NB), lambda i: (0, i)),
        out_shape=jax.ShapeDtypeStruct((4, N), jnp.float32),
    )(feats, dirT, w1s, w2s, w1r, w2r, w3r)


def kernel(position, direction, table, w_sig1, w_sig2, w_rgb1, w_rgb2,
           w_rgb3):
    px = position[:, 0]
    py = position[:, 1]
    pz = position[:, 2]
    t4 = table.reshape(N_LEVELS, T // 128, 128, F)
    b0 = lax.bitcast_convert_type(t4[..., 0].astype(jnp.bfloat16),
                                  jnp.uint16).astype(jnp.uint32)
    b1 = lax.bitcast_convert_type(t4[..., 1].astype(jnp.bfloat16),
                                  jnp.uint16).astype(jnp.uint32)
    tabp = lax.bitcast_convert_type(b0 | (b1 << 16),
                                    jnp.int32).reshape(N_LEVELS * T)
    N = position.shape[0]
    H = N // 2
    dirT = direction.T
    feats0 = _sc_hash_encode(px[:H], py[:H], pz[:H], tabp)
    feats1 = _sc_hash_encode(px[H:], py[H:], pz[H:], tabp)
    out0 = _tc_mlp(feats0, dirT[:, :H], w_sig1.T, w_sig2.T, w_rgb1.T,
                   w_rgb2.T, w_rgb3.T)
    out1 = _tc_mlp(feats1, dirT[:, H:], w_sig1.T, w_sig2.T, w_rgb1.T,
                   w_rgb2.T, w_rgb3.T)
    out4 = jnp.concatenate([out0, out1], axis=1)
    rgbs = out4[:3].T
    alphas = out4[3]
    return (rgbs, alphas)
